# baseline (speedup 1.0000x reference)
import jax
import jax.numpy as jnp
from jax import lax
from jax.experimental import pallas as pl
from jax.experimental.pallas import tpu as pltpu
from jax.experimental.pallas import tpu_sc as plsc

N = 10000
D = 128
H = 128
E = 320000
NC = 2
NS = 16
NW = NC * NS
CH = 64
NCHUNK = 160
HC = 40
EP = NW * NCHUNK * CH
N2 = 10240
RPT = N2 // NS
ZR = 32

_MESH = plsc.VectorSubcoreMesh(core_axis_name="c", subcore_axis_name="s")


def _sc_sum_body(x_hbm, src_hbm, dst_hbm, sums_out, src_idx, dst_idx,
                 r0, r1, r2, r3, sums_sp,
                 gs0, gs1, gs2, gs3, ss0, ss1, ss2, ss3):
    c = lax.axis_index("c")
    s = lax.axis_index("s")
    wid = c * NS + s
    base = s * RPT
    rows = [r0, r1, r2, r3]
    gsem = [gs0, gs1, gs2, gs3]
    ssem = [ss0, ss1, ss2, ss3]

    zeros16 = jnp.zeros((16,), jnp.float32)

    def zrow_body(i, _):
        r0[i // (D // 16), pl.ds((i % (D // 16)) * 16, 16)] = zeros16
        return 0
    lax.fori_loop(0, ZR * (D // 16), zrow_body, 0)

    def init_body(j, _):
        pltpu.sync_copy(r0.at[pl.ds(0, ZR)],
                        sums_sp.at[pl.ds(base + j * ZR, ZR)])
        return 0
    lax.fori_loop(0, RPT // ZR, init_body, 0)
    plsc.subcore_barrier()

    def gather(t, j):
        return pltpu.make_async_copy(x_hbm.at[src_idx.at[t]], rows[j],
                                     gsem[j])

    def scatter_start(t, j):
        pltpu.async_copy(rows[j], sums_sp.at[dst_idx.at[t]], ssem[j],
                         add=True)

    def scatter_wait(t, j):
        pltpu.make_async_copy(rows[j], sums_sp.at[dst_idx.at[t]],
                              ssem[j]).wait()

    for h in range(NCHUNK // HC):
        pltpu.sync_copy(src_hbm.at[wid, pl.ds(h * HC, HC)], src_idx)
        pltpu.sync_copy(dst_hbm.at[wid, pl.ds(h * HC, HC)], dst_idx)
        gather(0, 0).start()
        gather(1, 1).start()
        gather(2, 2).start()

        def quad_body(u, _):
            T = 4 * u
            for j in range(4):
                t = T + j
                gather(t, j).wait()
                scatter_start(t, j)
                nj = (j + 3) % 4

                @pl.when(t + 3 < HC)
                def _():
                    @pl.when(t >= 1)
                    def _():
                        scatter_wait(t - 1, nj)
                    gather(t + 3, nj).start()
            return 0
        lax.fori_loop(0, HC // 4, quad_body, 0)
        for k in range(4):
            scatter_wait(HC - 4 + k, k)
    plsc.subcore_barrier()

    pltpu.sync_copy(sums_sp.at[pl.ds(base, RPT)],
                    sums_out.at[c, pl.ds(base, RPT)])


_sc_sum = pl.kernel(
    _sc_sum_body,
    out_type=jax.ShapeDtypeStruct((NC, N2, D), jnp.float32),
    mesh=_MESH,
    scratch_types=[
        pltpu.VMEM((HC, CH), jnp.int32),
        pltpu.VMEM((HC, CH), jnp.int32),
        pltpu.VMEM((CH, D), jnp.float32),
        pltpu.VMEM((CH, D), jnp.float32),
        pltpu.VMEM((CH, D), jnp.float32),
        pltpu.VMEM((CH, D), jnp.float32),
        pltpu.VMEM_SHARED((N2, D), jnp.float32),
        pltpu.SemaphoreType.DMA,
        pltpu.SemaphoreType.DMA,
        pltpu.SemaphoreType.DMA,
        pltpu.SemaphoreType.DMA,
        pltpu.SemaphoreType.DMA,
        pltpu.SemaphoreType.DMA,
        pltpu.SemaphoreType.DMA,
        pltpu.SemaphoreType.DMA,
    ],
)


def _sc_cnt_body(dst_hbm, on_hbm, cnts_out, dst_idx, ones, zrow, cnts_sp,
                 cs0, cs1, cs2, cs3):
    c = lax.axis_index("c")
    s = lax.axis_index("s")
    wid = c * NS + s
    base = s * RPT
    csem = [cs0, cs1, cs2, cs3]

    zeros16 = jnp.zeros((16,), jnp.float32)

    def zrow_body(i, _):
        zrow[i // (D // 16), pl.ds((i % (D // 16)) * 16, 16)] = zeros16
        return 0
    lax.fori_loop(0, ZR * (D // 16), zrow_body, 0)

    def init_body(j, _):
        pltpu.sync_copy(zrow, cnts_sp.at[pl.ds(base + j * ZR, ZR)])
        return 0
    lax.fori_loop(0, RPT // ZR, init_body, 0)
    plsc.subcore_barrier()

    pltpu.sync_copy(on_hbm, ones)
    pltpu.sync_copy(dst_hbm.at[wid], dst_idx)

    def cnt_wait(t, j):
        pltpu.make_async_copy(ones, cnts_sp.at[dst_idx.at[t]],
                              csem[j]).wait()

    def quad_body(u, _):
        T = 4 * u
        for j in range(4):
            t = T + j

            @pl.when(t >= 4)
            def _():
                cnt_wait(t - 4, j)
            pltpu.async_copy(ones, cnts_sp.at[dst_idx.at[t]], csem[j],
                             add=True)
        return 0
    lax.fori_loop(0, NCHUNK // 4, quad_body, 0)
    for k in range(4):
        cnt_wait(NCHUNK - 4 + k, k)
    plsc.subcore_barrier()

    pltpu.sync_copy(cnts_sp.at[pl.ds(base, RPT)],
                    cnts_out.at[c, pl.ds(base, RPT)])


_sc_cnt = pl.kernel(
    _sc_cnt_body,
    out_type=jax.ShapeDtypeStruct((NC, N2, D), jnp.float32),
    mesh=_MESH,
    scratch_types=[
        pltpu.VMEM((NCHUNK, CH), jnp.int32),
        pltpu.VMEM((CH, D), jnp.float32),
        pltpu.VMEM((ZR, D), jnp.float32),
        pltpu.VMEM_SHARED((N2, D), jnp.float32),
        pltpu.SemaphoreType.DMA,
        pltpu.SemaphoreType.DMA,
        pltpu.SemaphoreType.DMA,
        pltpu.SemaphoreType.DMA,
    ],
)

_BM = 1000


def _tc_layer(ps, cnts, x, Wl, bl, Wr, relu):
    def body(ps_ref, cnt_ref, x_ref, wl_ref, bl_ref, wr_ref, o_ref):
        ssum = ps_ref[0] + ps_ref[1]
        cnt = cnt_ref[0, :, 0:1] + cnt_ref[1, :, 0:1]
        agg = ssum / jnp.maximum(cnt, 1.0)
        dn = (((1,), (1,)), ((), ()))
        out = (lax.dot_general(agg, wl_ref[...], dn,
                               preferred_element_type=jnp.float32)
               + lax.dot_general(x_ref[...], wr_ref[...], dn,
                                 preferred_element_type=jnp.float32)
               + bl_ref[...])
        if relu:
            out = jnp.maximum(out, 0.0)
        o_ref[...] = out

    return pl.pallas_call(
        body,
        grid=(N // _BM,),
        in_specs=[
            pl.BlockSpec((NC, _BM, D), lambda i: (0, i, 0)),
            pl.BlockSpec((NC, _BM, D), lambda i: (0, i, 0)),
            pl.BlockSpec((_BM, D), lambda i: (i, 0)),
            pl.BlockSpec((H, D), lambda i: (0, 0)),
            pl.BlockSpec((1, H), lambda i: (0, 0)),
            pl.BlockSpec((H, D), lambda i: (0, 0)),
        ],
        out_specs=pl.BlockSpec((_BM, H), lambda i: (i, 0)),
        out_shape=jax.ShapeDtypeStruct((N, H), jnp.float32),
    )(ps, cnts, x, Wl, bl, Wr)


def kernel(features, edge_index, W1l, b1l, W1r, W2l, b2l, W2r):
    pad = EP - E
    src_pad = jnp.arange(pad, dtype=jnp.int32) % N
    dst_pad = N + (jnp.arange(pad, dtype=jnp.int32) % (N2 - N))
    src = jnp.concatenate([edge_index[0], src_pad]).reshape(NW, NCHUNK, CH)
    dst = jnp.concatenate([edge_index[1], dst_pad]).reshape(NW, NCHUNK, CH)
    pc = _sc_cnt(dst, jnp.ones((CH, D), jnp.float32))
    ps1 = _sc_sum(features, src, dst)
    out1 = _tc_layer(ps1, pc, features, W1l, b1l.reshape(1, H), W1r, relu=True)
    ps2 = _sc_sum(out1, src, dst)
    out2 = _tc_layer(ps2, pc, out1, W2l, b2l.reshape(1, H), W2r, relu=False)
    return out2


# speedup vs baseline: 1.0181x; 1.0181x over previous
import jax
import jax.numpy as jnp
from jax import lax
from jax.experimental import pallas as pl
from jax.experimental.pallas import tpu as pltpu
from jax.experimental.pallas import tpu_sc as plsc

N = 10000
D = 128
H = 128
E = 320000
NC = 2
NS = 16
NW = NC * NS
CH = 128
NCHUNK = 80
HC = 40
CNW = 8
EP = NW * NCHUNK * CH
N2 = 10240
RPT = N2 // NS
ZR = 32

_MESH = plsc.VectorSubcoreMesh(core_axis_name="c", subcore_axis_name="s")


def _sc_sum_body(x_hbm, src_hbm, dst_hbm, sums_out, src_idx, dst_idx,
                 rows0, rows1, sums_sp, sem0, sem1):
    c = lax.axis_index("c")
    s = lax.axis_index("s")
    wid = c * NS + s
    base = s * RPT

    zeros16 = jnp.zeros((16,), jnp.float32)

    def zrow_body(i, _):
        rows0[i // (D // 16), pl.ds((i % (D // 16)) * 16, 16)] = zeros16
        return 0
    lax.fori_loop(0, ZR * (D // 16), zrow_body, 0)

    def init_body(j, _):
        pltpu.sync_copy(rows0.at[pl.ds(0, ZR)],
                        sums_sp.at[pl.ds(base + j * ZR, ZR)])
        return 0
    lax.fori_loop(0, RPT // ZR, init_body, 0)
    plsc.subcore_barrier()

    def gather(t, rows, sem):
        return pltpu.make_async_copy(x_hbm.at[src_idx.at[t]], rows, sem)

    for h in range(NCHUNK // HC):
        pltpu.sync_copy(src_hbm.at[wid, pl.ds(h * HC, HC)], src_idx)
        pltpu.sync_copy(dst_hbm.at[wid, pl.ds(h * HC, HC)], dst_idx)
        gather(0, rows0, sem0).start()

        def chunk_body(u, _):
            t0 = 2 * u
            gather(t0 + 1, rows1, sem1).start()
            gather(t0, rows0, sem0).wait()
            pltpu.sync_copy(rows0, sums_sp.at[dst_idx.at[t0]], add=True)

            @pl.when(t0 + 2 < HC)
            def _():
                gather(t0 + 2, rows0, sem0).start()
            gather(t0 + 1, rows1, sem1).wait()
            pltpu.sync_copy(rows1, sums_sp.at[dst_idx.at[t0 + 1]], add=True)
            return 0
        lax.fori_loop(0, HC // 2, chunk_body, 0)
    plsc.subcore_barrier()

    pltpu.sync_copy(sums_sp.at[pl.ds(base, RPT)],
                    sums_out.at[c, pl.ds(base, RPT)])


_sc_sum = pl.kernel(
    _sc_sum_body,
    out_type=jax.ShapeDtypeStruct((NC, N2, D), jnp.float32),
    mesh=_MESH,
    scratch_types=[
        pltpu.VMEM((HC, CH), jnp.int32),
        pltpu.VMEM((HC, CH), jnp.int32),
        pltpu.VMEM((CH, D), jnp.float32),
        pltpu.VMEM((CH, D), jnp.float32),
        pltpu.VMEM_SHARED((N2, D), jnp.float32),
        pltpu.SemaphoreType.DMA,
        pltpu.SemaphoreType.DMA,
    ],
)


def _sc_cnt_body(dst_hbm, on_hbm, cnts_out, dst_idx, ones, zrow, cnts_sp,
                 cs0, cs1, cs2, cs3):
    c = lax.axis_index("c")
    s = lax.axis_index("s")
    wid = c * NS + s
    base = s * RPT
    csem = [cs0, cs1, cs2, cs3]

    zeros16 = jnp.zeros((16,), jnp.float32)

    def zrow_body(i, _):
        zrow[i // (D // 16), pl.ds((i % (D // 16)) * 16, 16)] = zeros16
        return 0
    lax.fori_loop(0, ZR * (D // 16), zrow_body, 0)

    def init_body(j, _):
        pltpu.sync_copy(zrow, cnts_sp.at[pl.ds(base + j * ZR, ZR)])
        return 0
    lax.fori_loop(0, RPT // ZR, init_body, 0)
    plsc.subcore_barrier()

    pltpu.sync_copy(on_hbm, ones)
    pltpu.sync_copy(dst_hbm.at[wid], dst_idx)

    def cnt_wait(t, j):
        pltpu.make_async_copy(ones, cnts_sp.at[dst_idx.at[t]],
                              csem[j]).wait()

    def quad_body(u, _):
        T = 4 * u
        for j in range(4):
            t = T + j

            @pl.when(t >= 4)
            def _():
                cnt_wait(t - 4, j)
            pltpu.async_copy(ones, cnts_sp.at[dst_idx.at[t]], csem[j],
                             add=True)
        return 0
    lax.fori_loop(0, NCHUNK // 4, quad_body, 0)
    for k in range(4):
        cnt_wait(NCHUNK - 4 + k, k)
    plsc.subcore_barrier()

    pltpu.sync_copy(cnts_sp.at[pl.ds(base, RPT)],
                    cnts_out.at[c, pl.ds(base, RPT)])


_sc_cnt = pl.kernel(
    _sc_cnt_body,
    out_type=jax.ShapeDtypeStruct((NC, N2, D), jnp.float32),
    mesh=_MESH,
    scratch_types=[
        pltpu.VMEM((NCHUNK, CH), jnp.int32),
        pltpu.VMEM((CH, D), jnp.float32),
        pltpu.VMEM((ZR, D), jnp.float32),
        pltpu.VMEM_SHARED((N2, D), jnp.float32),
        pltpu.SemaphoreType.DMA,
        pltpu.SemaphoreType.DMA,
        pltpu.SemaphoreType.DMA,
        pltpu.SemaphoreType.DMA,
    ],
)

_BM = 1000


def _tc_layer(ps, cnts, x, Wl, bl, Wr, relu):
    def body(ps_ref, cnt_ref, x_ref, wl_ref, bl_ref, wr_ref, o_ref):
        ssum = ps_ref[0] + ps_ref[1]
        cnt = cnt_ref[0, :, 0:1] + cnt_ref[1, :, 0:1]
        agg = ssum / jnp.maximum(cnt, 1.0)
        dn = (((1,), (1,)), ((), ()))
        out = (lax.dot_general(agg, wl_ref[...], dn,
                               preferred_element_type=jnp.float32)
               + lax.dot_general(x_ref[...], wr_ref[...], dn,
                                 preferred_element_type=jnp.float32)
               + bl_ref[...])
        if relu:
            out = jnp.maximum(out, 0.0)
        o_ref[...] = out

    return pl.pallas_call(
        body,
        grid=(N // _BM,),
        in_specs=[
            pl.BlockSpec((NC, _BM, D), lambda i: (0, i, 0)),
            pl.BlockSpec((NC, _BM, D), lambda i: (0, i, 0)),
            pl.BlockSpec((_BM, D), lambda i: (i, 0)),
            pl.BlockSpec((H, D), lambda i: (0, 0)),
            pl.BlockSpec((1, H), lambda i: (0, 0)),
            pl.BlockSpec((H, D), lambda i: (0, 0)),
        ],
        out_specs=pl.BlockSpec((_BM, H), lambda i: (i, 0)),
        out_shape=jax.ShapeDtypeStruct((N, H), jnp.float32),
    )(ps, cnts, x, Wl, bl, Wr)


def kernel(features, edge_index, W1l, b1l, W1r, W2l, b2l, W2r):
    pad = EP - E
    src_pad = jnp.arange(pad, dtype=jnp.int32) % N
    dst_pad = N + (jnp.arange(pad, dtype=jnp.int32) % (N2 - N))
    src = jnp.concatenate([edge_index[0], src_pad]).reshape(NW, NCHUNK, CH)
    dst = jnp.concatenate([edge_index[1], dst_pad]).reshape(NW, NCHUNK, CH)
    ps1 = _sc_sum(features, src, dst)
    pc = _sc_cnt(dst, jnp.ones((CH, D), jnp.float32))
    out1 = _tc_layer(ps1, pc, features, W1l, b1l.reshape(1, H), W1r, relu=True)
    ps2 = _sc_sum(out1, src, dst)
    out2 = _tc_layer(ps2, pc, out1, W2l, b2l.reshape(1, H), W2r, relu=False)
    return out2


# speedup vs baseline: 1.2070x; 1.1855x over previous
import jax
import jax.numpy as jnp
from jax import lax
from jax.experimental import pallas as pl
from jax.experimental.pallas import tpu as pltpu
from jax.experimental.pallas import tpu_sc as plsc

N = 10000
D = 128
H = 128
E = 320000
NC = 2
NS = 16
NW = NC * NS
CH = 128
NCHUNK = 80
HC = 40
CW = 16
EP = NW * NCHUNK * CH
N2 = 10240
RPT = N2 // NS
ZR = 32

_MESH = plsc.VectorSubcoreMesh(core_axis_name="c", subcore_axis_name="s")


def _sc_sum_body(x_hbm, src_hbm, dst_hbm, sums_out, src_idx, dst_idx,
                 rows0, rows1, sums_sp, sem0, sem1):
    c = lax.axis_index("c")
    s = lax.axis_index("s")
    wid = c * NS + s
    base = s * RPT

    zeros16 = jnp.zeros((16,), jnp.float32)

    def zrow_body(i, _):
        rows0[i // (D // 16), pl.ds((i % (D // 16)) * 16, 16)] = zeros16
        return 0
    lax.fori_loop(0, ZR * (D // 16), zrow_body, 0)

    def init_body(j, _):
        pltpu.sync_copy(rows0.at[pl.ds(0, ZR)],
                        sums_sp.at[pl.ds(base + j * ZR, ZR)])
        return 0
    lax.fori_loop(0, RPT // ZR, init_body, 0)
    plsc.subcore_barrier()

    def gather(t, rows, sem):
        return pltpu.make_async_copy(x_hbm.at[src_idx.at[t]], rows, sem)

    for h in range(NCHUNK // HC):
        pltpu.sync_copy(src_hbm.at[wid, pl.ds(h * HC, HC)], src_idx)
        pltpu.sync_copy(dst_hbm.at[wid, pl.ds(h * HC, HC)], dst_idx)
        gather(0, rows0, sem0).start()

        def chunk_body(u, _):
            t0 = 2 * u
            gather(t0 + 1, rows1, sem1).start()
            gather(t0, rows0, sem0).wait()
            pltpu.sync_copy(rows0, sums_sp.at[dst_idx.at[t0]], add=True)

            @pl.when(t0 + 2 < HC)
            def _():
                gather(t0 + 2, rows0, sem0).start()
            gather(t0 + 1, rows1, sem1).wait()
            pltpu.sync_copy(rows1, sums_sp.at[dst_idx.at[t0 + 1]], add=True)
            return 0
        lax.fori_loop(0, HC // 2, chunk_body, 0)
    plsc.subcore_barrier()

    pltpu.sync_copy(sums_sp.at[pl.ds(base, RPT)],
                    sums_out.at[c, pl.ds(base, RPT)])


_sc_sum = pl.kernel(
    _sc_sum_body,
    out_type=jax.ShapeDtypeStruct((NC, N2, D), jnp.float32),
    mesh=_MESH,
    scratch_types=[
        pltpu.VMEM((HC, CH), jnp.int32),
        pltpu.VMEM((HC, CH), jnp.int32),
        pltpu.VMEM((CH, D), jnp.float32),
        pltpu.VMEM((CH, D), jnp.float32),
        pltpu.VMEM_SHARED((N2, D), jnp.float32),
        pltpu.SemaphoreType.DMA,
        pltpu.SemaphoreType.DMA,
    ],
)


def _sc_cnt_body(dst_hbm, zc_hbm, on_hbm, cnts_out, dst_idx, ones, zrow,
                 cnts_sp, cs0, cs1, cs2, cs3):
    c = lax.axis_index("c")
    s = lax.axis_index("s")
    wid = c * NS + s
    base = s * RPT
    csem = [cs0, cs1, cs2, cs3]

    pltpu.sync_copy(zc_hbm, zrow)
    pltpu.sync_copy(on_hbm, ones)

    def init_body(j, _):
        pltpu.sync_copy(zrow, cnts_sp.at[pl.ds(base + j * ZR, ZR)])
        return 0
    lax.fori_loop(0, RPT // ZR, init_body, 0)
    plsc.subcore_barrier()

    pltpu.sync_copy(dst_hbm.at[wid], dst_idx)

    def cnt_wait(t, j):
        pltpu.make_async_copy(ones, cnts_sp.at[dst_idx.at[t]],
                              csem[j]).wait()

    def quad_body(u, _):
        T = 4 * u
        for j in range(4):
            t = T + j

            @pl.when(t >= 4)
            def _():
                cnt_wait(t - 4, j)
            pltpu.async_copy(ones, cnts_sp.at[dst_idx.at[t]], csem[j],
                             add=True)
        return 0
    lax.fori_loop(0, NCHUNK // 4, quad_body, 0)
    for k in range(4):
        cnt_wait(NCHUNK - 4 + k, k)
    plsc.subcore_barrier()

    pltpu.sync_copy(cnts_sp.at[pl.ds(base, RPT)],
                    cnts_out.at[c, pl.ds(base, RPT)])


_sc_cnt = pl.kernel(
    _sc_cnt_body,
    out_type=jax.ShapeDtypeStruct((NC, N2, CW), jnp.float32),
    mesh=_MESH,
    compiler_params=pltpu.CompilerParams(use_tc_tiling_on_sc=False),
    scratch_types=[
        pltpu.VMEM((NCHUNK, CH), jnp.int32),
        pltpu.VMEM((CH, CW), jnp.float32),
        pltpu.VMEM((ZR, CW), jnp.float32),
        pltpu.VMEM_SHARED((N2, CW), jnp.float32),
        pltpu.SemaphoreType.DMA,
        pltpu.SemaphoreType.DMA,
        pltpu.SemaphoreType.DMA,
        pltpu.SemaphoreType.DMA,
    ],
)

_BM = 1000


def _tc_layer(ps, cnts, x, Wl, bl, Wr, relu):
    def body(ps_ref, cnt_ref, x_ref, wl_ref, bl_ref, wr_ref, o_ref):
        ssum = ps_ref[0] + ps_ref[1]
        cnt = cnt_ref[0, :, 0:1] + cnt_ref[1, :, 0:1]
        agg = ssum / jnp.maximum(cnt, 1.0)
        dn = (((1,), (1,)), ((), ()))
        out = (lax.dot_general(agg, wl_ref[...], dn,
                               preferred_element_type=jnp.float32)
               + lax.dot_general(x_ref[...], wr_ref[...], dn,
                                 preferred_element_type=jnp.float32)
               + bl_ref[...])
        if relu:
            out = jnp.maximum(out, 0.0)
        o_ref[...] = out

    return pl.pallas_call(
        body,
        grid=(N // _BM,),
        in_specs=[
            pl.BlockSpec((NC, _BM, D), lambda i: (0, i, 0)),
            pl.BlockSpec((NC, _BM, CW), lambda i: (0, i, 0)),
            pl.BlockSpec((_BM, D), lambda i: (i, 0)),
            pl.BlockSpec((H, D), lambda i: (0, 0)),
            pl.BlockSpec((1, H), lambda i: (0, 0)),
            pl.BlockSpec((H, D), lambda i: (0, 0)),
        ],
        out_specs=pl.BlockSpec((_BM, H), lambda i: (i, 0)),
        out_shape=jax.ShapeDtypeStruct((N, H), jnp.float32),
    )(ps, cnts, x, Wl, bl, Wr)


def kernel(features, edge_index, W1l, b1l, W1r, W2l, b2l, W2r):
    pad = EP - E
    src_pad = jnp.arange(pad, dtype=jnp.int32) % N
    dst_pad = N + (jnp.arange(pad, dtype=jnp.int32) % (N2 - N))
    src = jnp.concatenate([edge_index[0], src_pad]).reshape(NW, NCHUNK, CH)
    dst = jnp.concatenate([edge_index[1], dst_pad]).reshape(NW, NCHUNK, CH)
    ps1 = _sc_sum(features, src, dst)
    pc = _sc_cnt(dst, jnp.zeros((ZR, CW), jnp.float32),
                 jnp.ones((CH, CW), jnp.float32))
    out1 = _tc_layer(ps1, pc, features, W1l, b1l.reshape(1, H), W1r, relu=True)
    ps2 = _sc_sum(out1, src, dst)
    out2 = _tc_layer(ps2, pc, out1, W2l, b2l.reshape(1, H), W2r, relu=False)
    return out2


# speedup vs baseline: 1.2081x; 1.0009x over previous
import jax
import jax.numpy as jnp
from jax import lax
from jax.experimental import pallas as pl
from jax.experimental.pallas import tpu as pltpu
from jax.experimental.pallas import tpu_sc as plsc

N = 10000
D = 128
H = 128
E = 320000
NC = 2
NS = 16
NW = NC * NS
CH = 128
NCHUNK = 80
HC = 40
CW = 16
EP = NW * NCHUNK * CH
N2 = 10240
RPT = N2 // NS
ZR = 32

_MESH = plsc.VectorSubcoreMesh(core_axis_name="c", subcore_axis_name="s")


def _sc_sum_body(x_hbm, src_hbm, dst_hbm, sums_out, src_idx, dst_idx,
                 rows0, rows1, sums_sp, sem0, sem1):
    c = lax.axis_index("c")
    s = lax.axis_index("s")
    wid = c * NS + s
    base = s * RPT

    zeros16 = jnp.zeros((16,), jnp.float32)

    def zrow_body(i, _):
        rows0[i // (D // 16), pl.ds((i % (D // 16)) * 16, 16)] = zeros16
        return 0
    lax.fori_loop(0, ZR * (D // 16), zrow_body, 0)

    def init_body(j, _):
        pltpu.sync_copy(rows0.at[pl.ds(0, ZR)],
                        sums_sp.at[pl.ds(base + j * ZR, ZR)])
        return 0
    lax.fori_loop(0, RPT // ZR, init_body, 0)
    plsc.subcore_barrier()

    def gather(t, rows, sem):
        return pltpu.make_async_copy(x_hbm.at[src_idx.at[t]], rows, sem)

    for h in range(NCHUNK // HC):
        pltpu.sync_copy(src_hbm.at[wid, pl.ds(h * HC, HC)], src_idx)
        pltpu.sync_copy(dst_hbm.at[wid, pl.ds(h * HC, HC)], dst_idx)
        gather(0, rows0, sem0).start()

        def chunk_body(u, _):
            t0 = 2 * u
            gather(t0 + 1, rows1, sem1).start()
            gather(t0, rows0, sem0).wait()
            pltpu.sync_copy(rows0, sums_sp.at[dst_idx.at[t0]], add=True)

            @pl.when(t0 + 2 < HC)
            def _():
                gather(t0 + 2, rows0, sem0).start()
            gather(t0 + 1, rows1, sem1).wait()
            pltpu.sync_copy(rows1, sums_sp.at[dst_idx.at[t0 + 1]], add=True)
            return 0
        lax.fori_loop(0, HC // 2, chunk_body, 0)
    plsc.subcore_barrier()

    pltpu.sync_copy(sums_sp.at[pl.ds(base, RPT)],
                    sums_out.at[c, pl.ds(base, RPT)])


_sc_sum = pl.kernel(
    _sc_sum_body,
    out_type=jax.ShapeDtypeStruct((NC, N2, D), jnp.float32),
    mesh=_MESH,
    compiler_params=pltpu.CompilerParams(use_tc_tiling_on_sc=False),
    scratch_types=[
        pltpu.VMEM((HC, CH), jnp.int32),
        pltpu.VMEM((HC, CH), jnp.int32),
        pltpu.VMEM((CH, D), jnp.float32),
        pltpu.VMEM((CH, D), jnp.float32),
        pltpu.VMEM_SHARED((N2, D), jnp.float32),
        pltpu.SemaphoreType.DMA,
        pltpu.SemaphoreType.DMA,
    ],
)


def _sc_cnt_body(dst_hbm, zc_hbm, on_hbm, cnts_out, dst_idx, ones, zrow,
                 cnts_sp, cs0, cs1, cs2, cs3):
    c = lax.axis_index("c")
    s = lax.axis_index("s")
    wid = c * NS + s
    base = s * RPT
    csem = [cs0, cs1, cs2, cs3]

    pltpu.sync_copy(zc_hbm, zrow)
    pltpu.sync_copy(on_hbm, ones)

    def init_body(j, _):
        pltpu.sync_copy(zrow, cnts_sp.at[pl.ds(base + j * ZR, ZR)])
        return 0
    lax.fori_loop(0, RPT // ZR, init_body, 0)
    plsc.subcore_barrier()

    pltpu.sync_copy(dst_hbm.at[wid], dst_idx)

    def cnt_wait(t, j):
        pltpu.make_async_copy(ones, cnts_sp.at[dst_idx.at[t]],
                              csem[j]).wait()

    def quad_body(u, _):
        T = 4 * u
        for j in range(4):
            t = T + j

            @pl.when(t >= 4)
            def _():
                cnt_wait(t - 4, j)
            pltpu.async_copy(ones, cnts_sp.at[dst_idx.at[t]], csem[j],
                             add=True)
        return 0
    lax.fori_loop(0, NCHUNK // 4, quad_body, 0)
    for k in range(4):
        cnt_wait(NCHUNK - 4 + k, k)
    plsc.subcore_barrier()

    pltpu.sync_copy(cnts_sp.at[pl.ds(base, RPT)],
                    cnts_out.at[c, pl.ds(base, RPT)])


_sc_cnt = pl.kernel(
    _sc_cnt_body,
    out_type=jax.ShapeDtypeStruct((NC, N2, CW), jnp.float32),
    mesh=_MESH,
    compiler_params=pltpu.CompilerParams(use_tc_tiling_on_sc=False),
    scratch_types=[
        pltpu.VMEM((NCHUNK, CH), jnp.int32),
        pltpu.VMEM((CH, CW), jnp.float32),
        pltpu.VMEM((ZR, CW), jnp.float32),
        pltpu.VMEM_SHARED((N2, CW), jnp.float32),
        pltpu.SemaphoreType.DMA,
        pltpu.SemaphoreType.DMA,
        pltpu.SemaphoreType.DMA,
        pltpu.SemaphoreType.DMA,
    ],
)

_BM = 1000


def _tc_layer(ps, cnts, x, Wl, bl, Wr, relu):
    def body(ps_ref, cnt_ref, x_ref, wl_ref, bl_ref, wr_ref, o_ref):
        ssum = ps_ref[0] + ps_ref[1]
        cnt = cnt_ref[0, :, 0:1] + cnt_ref[1, :, 0:1]
        agg = ssum / jnp.maximum(cnt, 1.0)
        dn = (((1,), (1,)), ((), ()))
        out = (lax.dot_general(agg, wl_ref[...], dn,
                               preferred_element_type=jnp.float32)
               + lax.dot_general(x_ref[...], wr_ref[...], dn,
                                 preferred_element_type=jnp.float32)
               + bl_ref[...])
        if relu:
            out = jnp.maximum(out, 0.0)
        o_ref[...] = out

    return pl.pallas_call(
        body,
        grid=(N // _BM,),
        in_specs=[
            pl.BlockSpec((NC, _BM, D), lambda i: (0, i, 0)),
            pl.BlockSpec((NC, _BM, CW), lambda i: (0, i, 0)),
            pl.BlockSpec((_BM, D), lambda i: (i, 0)),
            pl.BlockSpec((H, D), lambda i: (0, 0)),
            pl.BlockSpec((1, H), lambda i: (0, 0)),
            pl.BlockSpec((H, D), lambda i: (0, 0)),
        ],
        out_specs=pl.BlockSpec((_BM, H), lambda i: (i, 0)),
        out_shape=jax.ShapeDtypeStruct((N, H), jnp.float32),
    )(ps, cnts, x, Wl, bl, Wr)


def kernel(features, edge_index, W1l, b1l, W1r, W2l, b2l, W2r):
    pad = EP - E
    src_pad = jnp.arange(pad, dtype=jnp.int32) % N
    dst_pad = N + (jnp.arange(pad, dtype=jnp.int32) % (N2 - N))
    src = jnp.concatenate([edge_index[0], src_pad]).reshape(NW, NCHUNK, CH)
    dst = jnp.concatenate([edge_index[1], dst_pad]).reshape(NW, NCHUNK, CH)
    ps1 = _sc_sum(features, src, dst)
    pc = _sc_cnt(dst, jnp.zeros((ZR, CW), jnp.float32),
                 jnp.ones((CH, CW), jnp.float32))
    out1 = _tc_layer(ps1, pc, features, W1l, b1l.reshape(1, H), W1r, relu=True)
    ps2 = _sc_sum(out1, src, dst)
    out2 = _tc_layer(ps2, pc, out1, W2l, b2l.reshape(1, H), W2r, relu=False)
    return out2
